# baseline (speedup 1.0000x reference)
import functools

import jax
import jax.numpy as jnp
from jax import lax
from jax.experimental import pallas as pl
from jax.experimental.pallas import tpu as pltpu
from jax.experimental.pallas import tpu_sc as plsc

EMB_S = 20
EMB_M = 20
EMB_H = 50
BATCH = 16384
CW = 16
KS = 2
KM = 2
KH = 4

_NC, _NS = 2, 16
_NW = _NC * _NS
_BPW = BATCH // _NW


@functools.cache
def _get_sc_gather():
  mesh = plsc.VectorSubcoreMesh(core_axis_name="c", subcore_axis_name="s",
                                num_cores=_NC, num_subcores=_NS)

  @functools.partial(
      pl.kernel,
      out_type=(
          jax.ShapeDtypeStruct((BATCH * KS, CW), jnp.float32),
          jax.ShapeDtypeStruct((BATCH * KM, CW), jnp.float32),
          jax.ShapeDtypeStruct((BATCH * KH, CW), jnp.float32),
      ),
      mesh=mesh,
      scratch_types=[
          pltpu.VMEM((_BPW * KS,), jnp.int32),
          pltpu.VMEM((_BPW * KM,), jnp.int32),
          pltpu.VMEM((_BPW * KH,), jnp.int32),
          pltpu.VMEM((_BPW * KS, CW), jnp.float32),
          pltpu.VMEM((_BPW * KM, CW), jnp.float32),
          pltpu.VMEM((_BPW * KH, CW), jnp.float32),
          pltpu.SemaphoreType.DMA,
      ],
      compiler_params=pltpu.CompilerParams(use_tc_tiling_on_sc=False),
  )
  def sc_gather(sc_hbm, mc_hbm, hc_hbm, ws_hbm, wm_hbm, wh_hbm,
                outs_hbm, outm_hbm, outh_hbm,
                sc_v, mc_v, hc_v, bs_v, bm_v, bh_v, sem):
    wid = lax.axis_index("s") * _NC + lax.axis_index("c")
    ns, nm, nh = _BPW * KS, _BPW * KM, _BPW * KH
    pltpu.sync_copy(sc_hbm.at[pl.ds(wid * ns, ns)], sc_v)
    pltpu.sync_copy(mc_hbm.at[pl.ds(wid * nm, nm)], mc_v)
    pltpu.sync_copy(hc_hbm.at[pl.ds(wid * nh, nh)], hc_v)
    gs = pltpu.async_copy(ws_hbm.at[sc_v], bs_v, sem)
    gm = pltpu.async_copy(wm_hbm.at[mc_v], bm_v, sem)
    gh = pltpu.async_copy(wh_hbm.at[hc_v], bh_v, sem)
    gs.wait()
    pltpu.sync_copy(bs_v, outs_hbm.at[pl.ds(wid * ns, ns)])
    gm.wait()
    pltpu.sync_copy(bm_v, outm_hbm.at[pl.ds(wid * nm, nm)])
    gh.wait()
    pltpu.sync_copy(bh_v, outh_hbm.at[pl.ds(wid * nh, nh)])

  return sc_gather


_TC_R = 512


def _tc_body(offs_ref, offm_ref, offh_ref, xs_ref, xm_ref, xh_ref, out_ref):
  offs = offs_ref[...]
  offm = offm_ref[...]
  offh = offh_ref[...]
  xs = xs_ref[...]
  xm = xm_ref[...]
  xh = xh_ref[...]
  sel_s = xs[:, 0:EMB_S]
  for k in range(4, 16, 4):
    sel_s = jnp.where(offs == k, xs[:, k:k + EMB_S], sel_s)
  sel_m = xm[:, 0:EMB_M]
  for k in range(4, 16, 4):
    sel_m = jnp.where(offm == k, xm[:, k:k + EMB_M], sel_m)
  sel_h = xh[:, 0:EMB_H]
  for k in range(2, 16, 2):
    sel_h = jnp.where(offh == k, xh[:, k:k + EMB_H], sel_h)
  out_ref[...] = jnp.concatenate([sel_s, sel_m, sel_h], axis=1)


@functools.cache
def _get_tc_compact():
  nb = BATCH // _TC_R
  return pl.pallas_call(
      _tc_body,
      grid=(nb,),
      in_specs=[
          pl.BlockSpec((_TC_R, 1), lambda i: (i, 0)),
          pl.BlockSpec((_TC_R, 1), lambda i: (i, 0)),
          pl.BlockSpec((_TC_R, 1), lambda i: (i, 0)),
          pl.BlockSpec((_TC_R, KS * CW), lambda i: (i, 0)),
          pl.BlockSpec((_TC_R, KM * CW), lambda i: (i, 0)),
          pl.BlockSpec((_TC_R, KH * CW), lambda i: (i, 0)),
      ],
      out_specs=pl.BlockSpec((_TC_R, EMB_S + EMB_M + EMB_H),
                             lambda i: (i, 0)),
      out_shape=jax.ShapeDtypeStruct((BATCH, EMB_S + EMB_M + EMB_H),
                                     jnp.float32),
  )


def kernel(store_idx, menu_idx, holiday_idx, W_store, W_menu, W_holiday):
  s = store_idx.astype(jnp.int32)
  m = menu_idx.astype(jnp.int32)
  h = holiday_idx.astype(jnp.int32)
  ws = EMB_S * s
  wm = EMB_M * m
  wh = EMB_H * h
  r2 = jnp.arange(KS, dtype=jnp.int32)
  r4 = jnp.arange(KH, dtype=jnp.int32)
  sc = ((ws >> 4)[:, None] + r2).reshape(-1)
  mc = ((wm >> 4)[:, None] + r2).reshape(-1)
  hc = ((wh >> 4)[:, None] + r4).reshape(-1)
  outs, outm, outh = _get_sc_gather()(
      sc, mc, hc,
      W_store.reshape(-1, CW), W_menu.reshape(-1, CW),
      W_holiday.reshape(-1, CW))
  return _get_tc_compact()(
      (ws & 15)[:, None], (wm & 15)[:, None], (wh & 15)[:, None],
      outs.reshape(BATCH, KS * CW), outm.reshape(BATCH, KM * CW),
      outh.reshape(BATCH, KH * CW))


# speedup vs baseline: 1.1657x; 1.1657x over previous
import functools

import jax
import jax.numpy as jnp
from jax import lax
from jax.experimental import pallas as pl
from jax.experimental.pallas import tpu as pltpu
from jax.experimental.pallas import tpu_sc as plsc

EMB_S = 20
EMB_M = 20
EMB_H = 50
BATCH = 16384
CW = 16
KS = 2
KH = 4

_NC, _NS = 2, 16
_NW = _NC * _NS
_BPW = BATCH // _NW


@functools.cache
def _get_sc_gather():
  mesh = plsc.VectorSubcoreMesh(core_axis_name="c", subcore_axis_name="s",
                                num_cores=_NC, num_subcores=_NS)

  @functools.partial(
      pl.kernel,
      out_type=(
          jax.ShapeDtypeStruct((BATCH, KS * CW), jnp.float32),
          jax.ShapeDtypeStruct((BATCH, KS * CW), jnp.float32),
          jax.ShapeDtypeStruct((BATCH, KH * CW), jnp.float32),
      ),
      mesh=mesh,
      scratch_types=(
          [pltpu.VMEM((_BPW,), jnp.int32) for _ in range(2 * KS + KH)]
          + [pltpu.VMEM((_BPW, CW), jnp.float32) for _ in range(2 * KS + KH)]
          + [pltpu.SemaphoreType.DMA]
      ),
      compiler_params=pltpu.CompilerParams(use_tc_tiling_on_sc=False),
  )
  def sc_gather(s0_hbm, s1_hbm, m0_hbm, m1_hbm, h0_hbm, h1_hbm, h2_hbm,
                h3_hbm, ws_hbm, wm_hbm, wh_hbm,
                outs_hbm, outm_hbm, outh_hbm,
                i0, i1, i2, i3, i4, i5, i6, i7,
                b0, b1, b2, b3, b4, b5, b6, b7, sem):
    wid = lax.axis_index("s") * _NC + lax.axis_index("c")
    base = wid * _BPW
    idx_hbms = (s0_hbm, s1_hbm, m0_hbm, m1_hbm, h0_hbm, h1_hbm, h2_hbm,
                h3_hbm)
    idxs = (i0, i1, i2, i3, i4, i5, i6, i7)
    bufs = (b0, b1, b2, b3, b4, b5, b6, b7)
    tabs = (ws_hbm, ws_hbm, wm_hbm, wm_hbm, wh_hbm, wh_hbm, wh_hbm, wh_hbm)
    for k in range(8):
      pltpu.sync_copy(idx_hbms[k].at[pl.ds(base, _BPW)], idxs[k])
    copies = [pltpu.async_copy(tabs[k].at[idxs[k]], bufs[k], sem)
              for k in range(8)]
    dsts = (
        outs_hbm.at[pl.ds(base, _BPW), pl.ds(0, CW)],
        outs_hbm.at[pl.ds(base, _BPW), pl.ds(CW, CW)],
        outm_hbm.at[pl.ds(base, _BPW), pl.ds(0, CW)],
        outm_hbm.at[pl.ds(base, _BPW), pl.ds(CW, CW)],
        outh_hbm.at[pl.ds(base, _BPW), pl.ds(0, CW)],
        outh_hbm.at[pl.ds(base, _BPW), pl.ds(CW, CW)],
        outh_hbm.at[pl.ds(base, _BPW), pl.ds(2 * CW, CW)],
        outh_hbm.at[pl.ds(base, _BPW), pl.ds(3 * CW, CW)],
    )
    for k in range(8):
      copies[k].wait()
      pltpu.sync_copy(bufs[k], dsts[k])

  return sc_gather


_TC_R = 512


def _tc_body(offs_ref, offm_ref, offh_ref, xs_ref, xm_ref, xh_ref, out_ref):
  offs = offs_ref[...]
  offm = offm_ref[...]
  offh = offh_ref[...]
  xs = xs_ref[...]
  xm = xm_ref[...]
  xh = xh_ref[...]
  sel_s = xs[:, 0:EMB_S]
  for k in range(4, 16, 4):
    sel_s = jnp.where(offs == k, xs[:, k:k + EMB_S], sel_s)
  sel_m = xm[:, 0:EMB_M]
  for k in range(4, 16, 4):
    sel_m = jnp.where(offm == k, xm[:, k:k + EMB_M], sel_m)
  sel_h = xh[:, 0:EMB_H]
  for k in range(2, 16, 2):
    sel_h = jnp.where(offh == k, xh[:, k:k + EMB_H], sel_h)
  out_ref[...] = jnp.concatenate([sel_s, sel_m, sel_h], axis=1)


@functools.cache
def _get_tc_compact():
  nb = BATCH // _TC_R
  return pl.pallas_call(
      _tc_body,
      grid=(nb,),
      in_specs=[
          pl.BlockSpec((_TC_R, 1), lambda i: (i, 0)),
          pl.BlockSpec((_TC_R, 1), lambda i: (i, 0)),
          pl.BlockSpec((_TC_R, 1), lambda i: (i, 0)),
          pl.BlockSpec((_TC_R, KS * CW), lambda i: (i, 0)),
          pl.BlockSpec((_TC_R, KS * CW), lambda i: (i, 0)),
          pl.BlockSpec((_TC_R, KH * CW), lambda i: (i, 0)),
      ],
      out_specs=pl.BlockSpec((_TC_R, EMB_S + EMB_M + EMB_H),
                             lambda i: (i, 0)),
      out_shape=jax.ShapeDtypeStruct((BATCH, EMB_S + EMB_M + EMB_H),
                                     jnp.float32),
  )


def kernel(store_idx, menu_idx, holiday_idx, W_store, W_menu, W_holiday):
  s = store_idx.astype(jnp.int32)
  m = menu_idx.astype(jnp.int32)
  h = holiday_idx.astype(jnp.int32)
  ws = EMB_S * s
  wm = EMB_M * m
  wh = EMB_H * h
  cs0 = ws >> 4
  cm0 = wm >> 4
  ch0 = wh >> 4
  outs, outm, outh = _get_sc_gather()(
      cs0, cs0 + 1, cm0, cm0 + 1, ch0, ch0 + 1, ch0 + 2, ch0 + 3,
      W_store.reshape(-1, CW), W_menu.reshape(-1, CW),
      W_holiday.reshape(-1, CW))
  return _get_tc_compact()(
      (ws & 15)[:, None], (wm & 15)[:, None], (wh & 15)[:, None],
      outs, outm, outh)


# speedup vs baseline: 1.4333x; 1.2296x over previous
import functools

import jax
import jax.numpy as jnp
from jax import lax
from jax.experimental import pallas as pl
from jax.experimental.pallas import tpu as pltpu
from jax.experimental.pallas import tpu_sc as plsc

EMB_S = 20
EMB_M = 20
EMB_H = 50
BATCH = 16384
CW = 16
KS = 2
KH = 4

_NC, _NS = 2, 16
_NW = _NC * _NS
_BPW = BATCH // _NW


@functools.cache
def _get_sc_gather():
  mesh = plsc.VectorSubcoreMesh(core_axis_name="c", subcore_axis_name="s",
                                num_cores=_NC, num_subcores=_NS)

  @functools.partial(
      pl.kernel,
      out_type=(
          jax.ShapeDtypeStruct((BATCH, KS * CW), jnp.float32),
          jax.ShapeDtypeStruct((BATCH, KS * CW), jnp.float32),
          jax.ShapeDtypeStruct((BATCH, KH * CW), jnp.float32),
      ),
      mesh=mesh,
      scratch_types=(
          [pltpu.VMEM((_BPW,), jnp.int32) for _ in range(2 * KS + KH)]
          + [pltpu.VMEM((_BPW, CW), jnp.float32) for _ in range(2 * KS + KH)]
          + [pltpu.SemaphoreType.DMA]
      ),
      compiler_params=pltpu.CompilerParams(use_tc_tiling_on_sc=False),
  )
  def sc_gather(s0_hbm, s1_hbm, m0_hbm, m1_hbm, h0_hbm, h1_hbm, h2_hbm,
                h3_hbm, ws_hbm, wm_hbm, wh_hbm,
                outs_hbm, outm_hbm, outh_hbm,
                i0, i1, i2, i3, i4, i5, i6, i7,
                b0, b1, b2, b3, b4, b5, b6, b7, sem):
    wid = lax.axis_index("s") * _NC + lax.axis_index("c")
    base = wid * _BPW
    idx_hbms = (s0_hbm, s1_hbm, m0_hbm, m1_hbm, h0_hbm, h1_hbm, h2_hbm,
                h3_hbm)
    idxs = (i0, i1, i2, i3, i4, i5, i6, i7)
    bufs = (b0, b1, b2, b3, b4, b5, b6, b7)
    tabs = (ws_hbm, ws_hbm, wm_hbm, wm_hbm, wh_hbm, wh_hbm, wh_hbm, wh_hbm)
    for k in range(8):
      pltpu.sync_copy(idx_hbms[k].at[pl.ds(base, _BPW)], idxs[k])
    copies = [pltpu.async_copy(tabs[k].at[idxs[k]], bufs[k], sem)
              for k in range(8)]
    dsts = (
        outs_hbm.at[pl.ds(base, _BPW), pl.ds(0, CW)],
        outs_hbm.at[pl.ds(base, _BPW), pl.ds(CW, CW)],
        outm_hbm.at[pl.ds(base, _BPW), pl.ds(0, CW)],
        outm_hbm.at[pl.ds(base, _BPW), pl.ds(CW, CW)],
        outh_hbm.at[pl.ds(base, _BPW), pl.ds(0, CW)],
        outh_hbm.at[pl.ds(base, _BPW), pl.ds(CW, CW)],
        outh_hbm.at[pl.ds(base, _BPW), pl.ds(2 * CW, CW)],
        outh_hbm.at[pl.ds(base, _BPW), pl.ds(3 * CW, CW)],
    )
    for k in range(8):
      copies[k].wait()
      pltpu.sync_copy(bufs[k], dsts[k])

  return sc_gather


_TC_R = 1024


def _tc_body(xs_ref, xm_ref, xh_ref, out_ref):
  out_ref[...] = jnp.concatenate(
      [xs_ref[:, 0:EMB_S], xm_ref[:, 0:EMB_M], xh_ref[:, 0:EMB_H]], axis=1)


@functools.cache
def _get_tc_compact():
  nb = BATCH // _TC_R
  return pl.pallas_call(
      _tc_body,
      grid=(nb,),
      in_specs=[
          pl.BlockSpec((_TC_R, KS * CW), lambda i: (i, 0)),
          pl.BlockSpec((_TC_R, KS * CW), lambda i: (i, 0)),
          pl.BlockSpec((_TC_R, KH * CW), lambda i: (i, 0)),
      ],
      out_specs=pl.BlockSpec((_TC_R, EMB_S + EMB_M + EMB_H),
                             lambda i: (i, 0)),
      out_shape=jax.ShapeDtypeStruct((BATCH, EMB_S + EMB_M + EMB_H),
                                     jnp.float32),
  )


def kernel(store_idx, menu_idx, holiday_idx, W_store, W_menu, W_holiday):
  s = store_idx.astype(jnp.int32)
  m = menu_idx.astype(jnp.int32)
  h = holiday_idx.astype(jnp.int32)
  ws_p = jnp.pad(W_store, ((0, 0), (0, KS * CW - EMB_S)))
  wm_p = jnp.pad(W_menu, ((0, 0), (0, KS * CW - EMB_M)))
  wh_p = jnp.pad(W_holiday, ((0, 0), (0, KH * CW - EMB_H)))
  cs0 = KS * s
  cm0 = KS * m
  ch0 = KH * h
  outs, outm, outh = _get_sc_gather()(
      cs0, cs0 + 1, cm0, cm0 + 1, ch0, ch0 + 1, ch0 + 2, ch0 + 3,
      ws_p.reshape(-1, CW), wm_p.reshape(-1, CW), wh_p.reshape(-1, CW))
  return _get_tc_compact()(outs, outm, outh)


# speedup vs baseline: 1.5079x; 1.0520x over previous
import functools

import jax
import jax.numpy as jnp
from jax import lax
from jax.experimental import pallas as pl
from jax.experimental.pallas import tpu as pltpu
from jax.experimental.pallas import tpu_sc as plsc

EMB_S = 20
EMB_M = 20
EMB_H = 50
EMB_T = EMB_S + EMB_M + EMB_H
BATCH = 16384
CW = 16
KS = 2
KH = 4

_NC, _NS = 2, 16
_NW = _NC * _NS
_BPW = BATCH // _NW


@functools.cache
def _get_sc_kernel():
  mesh = plsc.VectorSubcoreMesh(core_axis_name="c", subcore_axis_name="s",
                                num_cores=_NC, num_subcores=_NS)

  @functools.partial(
      pl.kernel,
      out_type=jax.ShapeDtypeStruct((BATCH * EMB_T,), jnp.float32),
      mesh=mesh,
      scratch_types=(
          [pltpu.VMEM((_BPW,), jnp.int32) for _ in range(2 * KS + KH)]
          + [pltpu.VMEM((_BPW, CW), jnp.float32) for _ in range(2 * KS + KH)]
          + [pltpu.VMEM((_BPW * EMB_T + CW,), jnp.float32),
             pltpu.SemaphoreType.DMA]
      ),
      compiler_params=pltpu.CompilerParams(use_tc_tiling_on_sc=False),
  )
  def sc_cat(s0_hbm, s1_hbm, m0_hbm, m1_hbm, h0_hbm, h1_hbm, h2_hbm,
             h3_hbm, ws_hbm, wm_hbm, wh_hbm, out_hbm,
             i0, i1, i2, i3, i4, i5, i6, i7,
             b0, b1, b2, b3, b4, b5, b6, b7, cat, sem):
    wid = lax.axis_index("s") * _NC + lax.axis_index("c")
    base = wid * _BPW
    idx_hbms = (s0_hbm, s1_hbm, m0_hbm, m1_hbm, h0_hbm, h1_hbm, h2_hbm,
                h3_hbm)
    idxs = (i0, i1, i2, i3, i4, i5, i6, i7)
    bufs = (b0, b1, b2, b3, b4, b5, b6, b7)
    tabs = (ws_hbm, ws_hbm, wm_hbm, wm_hbm, wh_hbm, wh_hbm, wh_hbm, wh_hbm)
    for k in range(8):
      pltpu.sync_copy(idx_hbms[k].at[pl.ds(base, _BPW)], idxs[k])
    copies = [pltpu.async_copy(tabs[k].at[idxs[k]], bufs[k], sem)
              for k in range(8)]
    for c in copies:
      c.wait()

    offs = (0, 16, EMB_S, EMB_S + 16,
            EMB_S + EMB_M, EMB_S + EMB_M + 16,
            EMB_S + EMB_M + 32, EMB_S + EMB_M + 48)

    def assemble(j, _):
      rb = EMB_T * j
      for k in range(8):
        cat[pl.ds(rb + offs[k], CW)] = bufs[k][j]
      return 0

    lax.fori_loop(0, _BPW, assemble, 0)
    pltpu.sync_copy(cat.at[pl.ds(0, _BPW * EMB_T)],
                    out_hbm.at[pl.ds(base * EMB_T, _BPW * EMB_T)])

  return sc_cat


def kernel(store_idx, menu_idx, holiday_idx, W_store, W_menu, W_holiday):
  s = store_idx.astype(jnp.int32)
  m = menu_idx.astype(jnp.int32)
  h = holiday_idx.astype(jnp.int32)
  ws_p = jnp.pad(W_store, ((0, 0), (0, KS * CW - EMB_S)))
  wm_p = jnp.pad(W_menu, ((0, 0), (0, KS * CW - EMB_M)))
  wh_p = jnp.pad(W_holiday, ((0, 0), (0, KH * CW - EMB_H)))
  cs0 = KS * s
  cm0 = KS * m
  ch0 = KH * h
  flat = _get_sc_kernel()(
      cs0, cs0 + 1, cm0, cm0 + 1, ch0, ch0 + 1, ch0 + 2, ch0 + 3,
      ws_p.reshape(-1, CW), wm_p.reshape(-1, CW), wh_p.reshape(-1, CW))
  return flat.reshape(BATCH, EMB_T)
